# baseline (speedup 1.0000x reference)
import functools

import jax
import jax.numpy as jnp
from jax import lax
from jax.experimental import pallas as pl
from jax.experimental.pallas import tpu as pltpu
from jax.experimental.pallas import tpu_sc as plsc

N = 10000
NP = 10240
E = 320000
NCORES = 2
NSUB = 16
NTILES = NCORES * NSUB
CHUNKS = 80
CW = 128
EP = NTILES * CHUNKS * CW
F_IN = 128
HID = 16
C_OUT = 16
SLICE_PER_SUB = NP // NSUB
NBUF = 8
DX = 24
FAST_C = 1
SLOW_C = 1 - FAST_C

_sc_mesh = plsc.VectorSubcoreMesh(core_axis_name="c", subcore_axis_name="s")
_sc_params = pltpu.CompilerParams(
    needs_layout_passes=False, use_tc_tiling_on_sc=False)


def _rsqrt16(d):
    i = jnp.int32(0x5F3759DF) - (plsc.bitcast(d, jnp.int32) >> 1)
    y = plsc.bitcast(i, jnp.float32)
    for _ in range(3):
        y = y * (1.5 - 0.5 * d * y * y)
    return y


def _zero_rows(ref, n):
    z16 = jnp.zeros((16,), jnp.float32)

    def body(i, _):
        ref[i] = z16
        return 0

    lax.fori_loop(0, n, body, 0)


def _zero_flat(ref, n16):
    z16 = jnp.zeros((16,), jnp.float32)

    def body(i, _):
        ref[pl.ds(i * 16, 16)] = z16
        return 0

    lax.fori_loop(0, n16, body, 0)


def _agg_ring(colb, wb, row_of, nch, v_hbm, sh_agg, rows_v, sems):
    for b in range(NBUF):
        pltpu.async_copy(v_hbm.at[colb.at[b]], rows_v.at[b], sems.at[b])

    def body(i, _):
        for b in range(NBUF):
            jj = i * NBUF + b
            pltpu.make_async_copy(
                v_hbm.at[colb.at[jj]], rows_v.at[b], sems.at[b]).wait()
            for k in range(CW // 16):
                w16 = wb[jj, pl.ds(k * 16, 16)]
                base = k * 16
                for g in range(16):
                    wg = jnp.broadcast_to(w16[g], (16,))
                    rows_v[b, base + g] = rows_v[b, base + g] * wg
            pltpu.sync_copy(rows_v.at[b], sh_agg.at[row_of(jj)], add=True)

            @pl.when(jj + NBUF < nch)
            def _():
                pltpu.async_copy(
                    v_hbm.at[colb.at[jj + NBUF]], rows_v.at[b], sems.at[b])
        return 0

    lax.fori_loop(0, nch // NBUF, body, 0)


def _agg_writeout(c, s, sh_agg, agg_hbm):
    def body(j, _):
        r0 = s * SLICE_PER_SUB + j * CW
        pltpu.sync_copy(sh_agg.at[pl.ds(r0, CW)], agg_hbm.at[c, pl.ds(r0, CW)])
        return 0

    lax.fori_loop(0, SLICE_PER_SUB // CW, body, 0)


@functools.partial(
    pl.kernel,
    out_type=jax.ShapeDtypeStruct((NTILES, CHUNKS, CW), jnp.float32),
    mesh=_sc_mesh,
    compiler_params=_sc_params,
    scratch_types=(
        pltpu.VMEM((2, CHUNKS, CW), jnp.int32),
        pltpu.VMEM((2, CHUNKS, CW), jnp.float32),
        pltpu.VMEM((CHUNKS, CW), jnp.int32),
        pltpu.VMEM((CHUNKS, CW), jnp.float32),
        pltpu.VMEM((NP,), jnp.float32),
        pltpu.VMEM((NP,), jnp.float32),
        pltpu.VMEM((SLICE_PER_SUB,), jnp.float32),
        pltpu.VMEM((SLICE_PER_SUB,), jnp.float32),
        pltpu.VMEM((SLICE_PER_SUB,), jnp.float32),
        pltpu.VMEM_SHARED((NSUB, NP), jnp.float32),
        pltpu.VMEM_SHARED((NP,), jnp.float32),
    ),
)
def _sc_prep(row_hbm, col_hbm, ew_hbm, wn_hbm,
             row2, ew2, colb, wb, degl, disb, acc, tmpd, disc,
             sh_slots, sh_dis):
    c = lax.axis_index("c")
    s = lax.axis_index("s")
    wid = c * NSUB + s

    pltpu.sync_copy(row_hbm.at[s], row2.at[0])
    pltpu.sync_copy(row_hbm.at[s + NSUB], row2.at[1])
    pltpu.sync_copy(ew_hbm.at[s], ew2.at[0])
    pltpu.sync_copy(ew_hbm.at[s + NSUB], ew2.at[1])
    pltpu.sync_copy(col_hbm.at[wid], colb)

    _zero_flat(degl, NP // 16)

    def deg_body(j, _):
        for t in range(2):
            for k in range(8):
                sl = pl.ds(k * 16, 16)
                plsc.addupdate_scatter(degl, [row2[t, j, sl]], ew2[t, j, sl])
        return 0

    lax.fori_loop(0, CHUNKS, deg_body, 0)

    pltpu.sync_copy(degl, sh_slots.at[s])
    plsc.subcore_barrier()

    base = s * SLICE_PER_SUB
    _zero_flat(acc, SLICE_PER_SUB // 16)

    def red_body(t, _):
        pltpu.sync_copy(sh_slots.at[t, pl.ds(base, SLICE_PER_SUB)], tmpd)

        def add_body(r, _):
            sl = pl.ds(r * 16, 16)
            acc[sl] = acc[sl] + tmpd[sl]
            return 0

        lax.fori_loop(0, SLICE_PER_SUB // 16, add_body, 0)
        return 0

    lax.fori_loop(0, NSUB, red_body, 0)

    def dis_body(r, _):
        sl = pl.ds(r * 16, 16)
        d = acc[sl]
        y = _rsqrt16(jnp.maximum(d, 1e-30))
        disc[sl] = jnp.where(d > 0, y, 0.0)
        return 0

    lax.fori_loop(0, SLICE_PER_SUB // 16, dis_body, 0)
    pltpu.sync_copy(disc, sh_dis.at[pl.ds(base, SLICE_PER_SUB)])
    plsc.subcore_barrier()
    pltpu.sync_copy(sh_dis, disb)

    def wn_body(j, _):
        for k in range(8):
            sl = pl.ds(k * 16, 16)
            dr = plsc.load_gather(disb, [row2[c, j, sl]])
            dc = plsc.load_gather(disb, [colb[j, sl]])
            wb[j, sl] = dr * ew2[c, j, sl] * dc
        return 0

    lax.fori_loop(0, CHUNKS, wn_body, 0)
    pltpu.sync_copy(wb, wn_hbm.at[wid])


@functools.partial(
    pl.kernel,
    out_type=jax.ShapeDtypeStruct((NCORES, NP, HID), jnp.float32),
    mesh=_sc_mesh,
    compiler_params=_sc_params,
    scratch_types=(
        pltpu.VMEM((CHUNKS, CW), jnp.int32),
        pltpu.VMEM((CHUNKS, CW), jnp.int32),
        pltpu.VMEM((CHUNKS, CW), jnp.float32),
        pltpu.VMEM((DX, CW), jnp.int32),
        pltpu.VMEM((DX, CW), jnp.int32),
        pltpu.VMEM((DX, CW), jnp.float32),
        pltpu.VMEM((CW, 16), jnp.float32),
        pltpu.VMEM((NBUF, CW, HID), jnp.float32),
        pltpu.VMEM_SHARED((NP, HID), jnp.float32),
        pltpu.SemaphoreType.DMA((NBUF,)),
    ),
)
def _sc_agg(row_hbm, col_hbm, wn_hbm, v_hbm, agg_hbm,
            rowb, colb, wb, rowx, colx, wx, zb, rows_v, sh_agg, sems):
    c = lax.axis_index("c")
    s = lax.axis_index("s")
    wid = c * NSUB + s

    pltpu.sync_copy(row_hbm.at[wid], rowb)
    pltpu.sync_copy(col_hbm.at[wid], colb)
    pltpu.sync_copy(wn_hbm.at[wid], wb)
    _zero_rows(zb, CW)

    def za_body(j, _):
        pltpu.sync_copy(zb, sh_agg.at[pl.ds(s * SLICE_PER_SUB + j * CW, CW)])
        return 0

    lax.fori_loop(0, SLICE_PER_SUB // CW, za_body, 0)
    plsc.subcore_barrier()

    nch = jnp.where(c == SLOW_C, CHUNKS - DX, CHUNKS)
    _agg_ring(colb, wb, lambda jj: rowb.at[jj], nch,
              v_hbm, sh_agg, rows_v, sems)

    @pl.when(c == FAST_C)
    def _():
        other = SLOW_C * NSUB + s
        sl = pl.ds(CHUNKS - DX, DX)
        pltpu.sync_copy(row_hbm.at[other, sl], rowx)
        pltpu.sync_copy(col_hbm.at[other, sl], colx)
        pltpu.sync_copy(wn_hbm.at[other, sl], wx)
        _agg_ring(colx, wx, lambda jj: rowx.at[jj], DX,
                  v_hbm, sh_agg, rows_v, sems)

    plsc.subcore_barrier()
    _agg_writeout(c, s, sh_agg, agg_hbm)


def _mm1_body(x_ref, w0_ref, w1_ref, y0_ref, y1_ref):
    x = x_ref[...]
    y0_ref[...] = jnp.dot(x, w0_ref[...], preferred_element_type=jnp.float32)
    y1_ref[...] = jnp.dot(x, w1_ref[...], preferred_element_type=jnp.float32)


def _mid_body(y0_ref, agg_ref, b_ref, w0_ref, w1_ref, z0_ref, z1_ref):
    p = agg_ref[0] + agg_ref[1]
    h = jnp.maximum(y0_ref[...] - p + b_ref[0:1, :], 0.0)
    z0_ref[...] = jnp.dot(h, w0_ref[...], preferred_element_type=jnp.float32)
    z1_ref[...] = jnp.dot(h, w1_ref[...], preferred_element_type=jnp.float32)


def _fin_body(z0_ref, agg_ref, b_ref, out_ref):
    o = z0_ref[...] - (agg_ref[0] + agg_ref[1]) + b_ref[0:1, :]
    m = jnp.max(o, axis=1, keepdims=True)
    ex = jnp.exp(o - m)
    out_ref[...] = o - m - jnp.log(jnp.sum(ex, axis=1, keepdims=True))


_RB = 1000


def kernel(x, edge_index, edge_weight, W1, b1, W2, b2):
    row = edge_index[0]
    col = edge_index[1]
    pad = EP - E
    zpad_i = jnp.zeros((pad,), row.dtype)
    rowp = jnp.concatenate([row, zpad_i]).reshape(NTILES, CHUNKS, CW)
    colp = jnp.concatenate([col, zpad_i]).reshape(NTILES, CHUNKS, CW)
    ewp = jnp.concatenate([edge_weight, jnp.zeros((pad,), edge_weight.dtype)])
    ewp = ewp.reshape(NTILES, CHUNKS, CW)
    b1b = jnp.broadcast_to(b1.reshape(1, HID), (8, HID))
    b2b = jnp.broadcast_to(b2.reshape(1, C_OUT), (8, C_OUT))

    grid = N // _RB
    y0, y1 = pl.pallas_call(
        _mm1_body,
        grid=(grid,),
        in_specs=[
            pl.BlockSpec((_RB, F_IN), lambda i: (i, 0)),
            pl.BlockSpec((F_IN, HID), lambda i: (0, 0)),
            pl.BlockSpec((F_IN, HID), lambda i: (0, 0)),
        ],
        out_specs=[
            pl.BlockSpec((_RB, HID), lambda i: (i, 0)),
            pl.BlockSpec((_RB, HID), lambda i: (i, 0)),
        ],
        out_shape=[
            jax.ShapeDtypeStruct((N, HID), jnp.float32),
            jax.ShapeDtypeStruct((N, HID), jnp.float32),
        ],
    )(x, W1[0], W1[1])

    wn = _sc_prep(rowp, colp, ewp)
    agg1 = _sc_agg(rowp, colp, wn, y1)

    z0, z1 = pl.pallas_call(
        _mid_body,
        grid=(grid,),
        in_specs=[
            pl.BlockSpec((_RB, HID), lambda i: (i, 0)),
            pl.BlockSpec((NCORES, _RB, HID), lambda i: (0, i, 0)),
            pl.BlockSpec((8, HID), lambda i: (0, 0)),
            pl.BlockSpec((HID, C_OUT), lambda i: (0, 0)),
            pl.BlockSpec((HID, C_OUT), lambda i: (0, 0)),
        ],
        out_specs=[
            pl.BlockSpec((_RB, C_OUT), lambda i: (i, 0)),
            pl.BlockSpec((_RB, C_OUT), lambda i: (i, 0)),
        ],
        out_shape=[
            jax.ShapeDtypeStruct((N, C_OUT), jnp.float32),
            jax.ShapeDtypeStruct((N, C_OUT), jnp.float32),
        ],
    )(y0, agg1, b1b, W2[0], W2[1])

    agg2 = _sc_agg(rowp, colp, wn, z1)

    out = pl.pallas_call(
        _fin_body,
        grid=(grid,),
        in_specs=[
            pl.BlockSpec((_RB, C_OUT), lambda i: (i, 0)),
            pl.BlockSpec((NCORES, _RB, C_OUT), lambda i: (0, i, 0)),
            pl.BlockSpec((8, C_OUT), lambda i: (0, 0)),
        ],
        out_specs=pl.BlockSpec((_RB, C_OUT), lambda i: (i, 0)),
        out_shape=jax.ShapeDtypeStruct((N, C_OUT), jnp.float32),
    )(z0, agg2, b2b)
    return out


# speedup vs baseline: 1.2580x; 1.2580x over previous
import functools

import jax
import jax.numpy as jnp
from jax import lax
from jax.experimental import pallas as pl
from jax.experimental.pallas import tpu as pltpu
from jax.experimental.pallas import tpu_sc as plsc

N = 10000
NP = 10240
E = 320000
NCORES = 2
NSUB = 16
NTILES = NCORES * NSUB
CHUNKS = 80
CW = 128
EP = NTILES * CHUNKS * CW
F_IN = 128
HID = 16
C_OUT = 16
SLICE_PER_SUB = NP // NSUB
NBUF = 8

_sc_mesh = plsc.VectorSubcoreMesh(core_axis_name="c", subcore_axis_name="s")
_sc_params = pltpu.CompilerParams(
    needs_layout_passes=False, use_tc_tiling_on_sc=False)


def _rsqrt16(d):
    i = jnp.int32(0x5F3759DF) - (plsc.bitcast(d, jnp.int32) >> 1)
    y = plsc.bitcast(i, jnp.float32)
    for _ in range(3):
        y = y * (1.5 - 0.5 * d * y * y)
    return y


def _zero_rows(ref, n):
    z16 = jnp.zeros((16,), jnp.float32)

    def body(i, _):
        ref[i] = z16
        return 0

    lax.fori_loop(0, n, body, 0)


def _zero_flat(ref, n16):
    z16 = jnp.zeros((16,), jnp.float32)

    def body(i, _):
        ref[pl.ds(i * 16, 16)] = z16
        return 0

    lax.fori_loop(0, n16, body, 0)


def _agg_ring(colb, wb, row_of, nch, v_hbm, sh_agg, rows_v, sems):
    for b in range(NBUF):
        pltpu.async_copy(v_hbm.at[colb.at[b]], rows_v.at[b], sems.at[b])

    def body(i, _):
        for b in range(NBUF):
            jj = i * NBUF + b
            pltpu.make_async_copy(
                v_hbm.at[colb.at[jj]], rows_v.at[b], sems.at[b]).wait()
            for k in range(CW // 16):
                w16 = wb[jj, pl.ds(k * 16, 16)]
                base = k * 16
                for g in range(16):
                    wg = jnp.broadcast_to(w16[g], (16,))
                    rows_v[b, base + g] = rows_v[b, base + g] * wg
            pltpu.sync_copy(rows_v.at[b], sh_agg.at[row_of(jj)], add=True)

            @pl.when(jj + NBUF < nch)
            def _():
                pltpu.async_copy(
                    v_hbm.at[colb.at[jj + NBUF]], rows_v.at[b], sems.at[b])
        return 0

    lax.fori_loop(0, nch // NBUF, body, 0)


def _agg_writeout(c, s, sh_agg, agg_hbm):
    def body(j, _):
        r0 = s * SLICE_PER_SUB + j * CW
        pltpu.sync_copy(sh_agg.at[pl.ds(r0, CW)], agg_hbm.at[c, pl.ds(r0, CW)])
        return 0

    lax.fori_loop(0, SLICE_PER_SUB // CW, body, 0)


@functools.partial(
    pl.kernel,
    out_type=jax.ShapeDtypeStruct((NTILES, CHUNKS, CW), jnp.float32),
    mesh=_sc_mesh,
    compiler_params=_sc_params,
    scratch_types=(
        pltpu.VMEM((2, CHUNKS, CW), jnp.int32),
        pltpu.VMEM((2, CHUNKS, CW), jnp.float32),
        pltpu.VMEM((CHUNKS, CW), jnp.int32),
        pltpu.VMEM((CHUNKS, CW), jnp.float32),
        pltpu.VMEM((NP,), jnp.float32),
        pltpu.VMEM((NP,), jnp.float32),
        pltpu.VMEM((SLICE_PER_SUB,), jnp.float32),
        pltpu.VMEM((SLICE_PER_SUB,), jnp.float32),
        pltpu.VMEM((SLICE_PER_SUB,), jnp.float32),
        pltpu.VMEM_SHARED((NSUB, NP), jnp.float32),
        pltpu.VMEM_SHARED((NP,), jnp.float32),
    ),
)
def _sc_prep(row_hbm, col_hbm, ew_hbm, wn_hbm,
             row2, ew2, colb, wb, degl, disb, acc, tmpd, disc,
             sh_slots, sh_dis):
    c = lax.axis_index("c")
    s = lax.axis_index("s")
    wid = c * NSUB + s

    pltpu.sync_copy(row_hbm.at[s], row2.at[0])
    pltpu.sync_copy(row_hbm.at[s + NSUB], row2.at[1])
    pltpu.sync_copy(ew_hbm.at[s], ew2.at[0])
    pltpu.sync_copy(ew_hbm.at[s + NSUB], ew2.at[1])
    pltpu.sync_copy(col_hbm.at[wid], colb)

    _zero_flat(degl, NP // 16)

    def deg_body(j, _):
        for t in range(2):
            for k in range(8):
                sl = pl.ds(k * 16, 16)
                plsc.addupdate_scatter(degl, [row2[t, j, sl]], ew2[t, j, sl])
        return 0

    lax.fori_loop(0, CHUNKS, deg_body, 0)

    pltpu.sync_copy(degl, sh_slots.at[s])
    plsc.subcore_barrier()

    base = s * SLICE_PER_SUB
    _zero_flat(acc, SLICE_PER_SUB // 16)

    def red_body(t, _):
        pltpu.sync_copy(sh_slots.at[t, pl.ds(base, SLICE_PER_SUB)], tmpd)

        def add_body(r, _):
            sl = pl.ds(r * 16, 16)
            acc[sl] = acc[sl] + tmpd[sl]
            return 0

        lax.fori_loop(0, SLICE_PER_SUB // 16, add_body, 0)
        return 0

    lax.fori_loop(0, NSUB, red_body, 0)

    def dis_body(r, _):
        sl = pl.ds(r * 16, 16)
        d = acc[sl]
        y = _rsqrt16(jnp.maximum(d, 1e-30))
        disc[sl] = jnp.where(d > 0, y, 0.0)
        return 0

    lax.fori_loop(0, SLICE_PER_SUB // 16, dis_body, 0)
    pltpu.sync_copy(disc, sh_dis.at[pl.ds(base, SLICE_PER_SUB)])
    plsc.subcore_barrier()
    pltpu.sync_copy(sh_dis, disb)

    def wn_body(j, _):
        for k in range(8):
            sl = pl.ds(k * 16, 16)
            dr = plsc.load_gather(disb, [row2[c, j, sl]])
            dc = plsc.load_gather(disb, [colb[j, sl]])
            wb[j, sl] = dr * ew2[c, j, sl] * dc
        return 0

    lax.fori_loop(0, CHUNKS, wn_body, 0)
    pltpu.sync_copy(wb, wn_hbm.at[wid])


@functools.partial(
    pl.kernel,
    out_type=jax.ShapeDtypeStruct((NCORES, NP, HID), jnp.float32),
    mesh=_sc_mesh,
    compiler_params=_sc_params,
    scratch_types=(
        pltpu.VMEM((CHUNKS, CW), jnp.int32),
        pltpu.VMEM((CHUNKS, CW), jnp.int32),
        pltpu.VMEM((CHUNKS, CW), jnp.float32),
        pltpu.VMEM((CW, 16), jnp.float32),
        pltpu.VMEM((NBUF, CW, HID), jnp.float32),
        pltpu.VMEM_SHARED((NP, HID), jnp.float32),
        pltpu.SemaphoreType.DMA((NBUF,)),
    ),
)
def _sc_agg(row_hbm, col_hbm, wn_hbm, v_hbm, agg_hbm,
            rowb, colb, wb, zb, rows_v, sh_agg, sems):
    c = lax.axis_index("c")
    s = lax.axis_index("s")
    wid = c * NSUB + s

    pltpu.sync_copy(row_hbm.at[wid], rowb)
    pltpu.sync_copy(col_hbm.at[wid], colb)
    pltpu.sync_copy(wn_hbm.at[wid], wb)
    _zero_rows(zb, CW)

    def za_body(j, _):
        pltpu.sync_copy(zb, sh_agg.at[pl.ds(s * SLICE_PER_SUB + j * CW, CW)])
        return 0

    lax.fori_loop(0, SLICE_PER_SUB // CW, za_body, 0)
    plsc.subcore_barrier()

    _agg_ring(colb, wb, lambda jj: rowb.at[jj], CHUNKS,
              v_hbm, sh_agg, rows_v, sems)
    plsc.subcore_barrier()
    _agg_writeout(c, s, sh_agg, agg_hbm)


def _mm1_body(x_ref, w0_ref, w1_ref, y0_ref, y1_ref):
    x = x_ref[...]
    y0_ref[...] = jnp.dot(x, w0_ref[...], preferred_element_type=jnp.float32)
    y1_ref[...] = jnp.dot(x, w1_ref[...], preferred_element_type=jnp.float32)


def _mid_body(y0_ref, agg_ref, b_ref, w0_ref, w1_ref, z0_ref, z1_ref):
    h = jnp.maximum(y0_ref[...] - (agg_ref[0] + agg_ref[1]) + b_ref[0:1, :],
                    0.0)
    z0_ref[...] = jnp.dot(h, w0_ref[...], preferred_element_type=jnp.float32)
    z1_ref[...] = jnp.dot(h, w1_ref[...], preferred_element_type=jnp.float32)


def _fin_body(z0_ref, agg_ref, b_ref, sg_ref, out_ref):
    o = z0_ref[...] - (agg_ref[0] + agg_ref[1]) + b_ref[0:1, :]
    m = jnp.max(o, axis=1, keepdims=True)
    ex = jnp.exp(o - m)
    gs = jnp.dot(ex, sg_ref[...], preferred_element_type=jnp.float32)
    out_ref[...] = o - m - jnp.log(gs)


_RB = 1024


def kernel(x, edge_index, edge_weight, W1, b1, W2, b2):
    row = edge_index[0]
    col = edge_index[1]
    pad = EP - E
    zpad_i = jnp.zeros((pad,), row.dtype)
    rowp = jnp.concatenate([row, zpad_i]).reshape(NTILES, CHUNKS, CW)
    colp = jnp.concatenate([col, zpad_i]).reshape(NTILES, CHUNKS, CW)
    ewp = jnp.concatenate([edge_weight, jnp.zeros((pad,), edge_weight.dtype)])
    ewp = ewp.reshape(NTILES, CHUNKS, CW)
    b1p = jnp.broadcast_to(jnp.tile(b1, 8).reshape(1, 128), (8, 128))
    b2p = jnp.broadcast_to(jnp.tile(b2, 8).reshape(1, 128), (8, 128))
    eye8 = jnp.eye(8, dtype=jnp.float32)
    bd20 = jnp.kron(eye8, W2[0])
    bd21 = jnp.kron(eye8, W2[1])
    sg = jnp.kron(eye8, jnp.ones((C_OUT, C_OUT), jnp.float32))
    xp = jnp.pad(x, ((0, NP - N), (0, 0)))

    grid = NP // _RB
    y0, y1 = pl.pallas_call(
        _mm1_body,
        grid=(grid,),
        in_specs=[
            pl.BlockSpec((_RB, F_IN), lambda i: (i, 0)),
            pl.BlockSpec((F_IN, HID), lambda i: (0, 0)),
            pl.BlockSpec((F_IN, HID), lambda i: (0, 0)),
        ],
        out_specs=[
            pl.BlockSpec((_RB, HID), lambda i: (i, 0)),
            pl.BlockSpec((_RB, HID), lambda i: (i, 0)),
        ],
        out_shape=[
            jax.ShapeDtypeStruct((NP, HID), jnp.float32),
            jax.ShapeDtypeStruct((NP, HID), jnp.float32),
        ],
    )(xp, W1[0], W1[1])

    wn = _sc_prep(rowp, colp, ewp)
    agg1 = _sc_agg(rowp, colp, wn, y1)

    PK = NP * HID // 128
    _PB = _RB * HID // 128
    agg1p = agg1.reshape(NCORES, PK, 128)
    y0p = y0.reshape(PK, 128)

    z0p, z1p = pl.pallas_call(
        _mid_body,
        grid=(grid,),
        in_specs=[
            pl.BlockSpec((_PB, 128), lambda i: (i, 0)),
            pl.BlockSpec((NCORES, _PB, 128), lambda i: (0, i, 0)),
            pl.BlockSpec((8, 128), lambda i: (0, 0)),
            pl.BlockSpec((128, 128), lambda i: (0, 0)),
            pl.BlockSpec((128, 128), lambda i: (0, 0)),
        ],
        out_specs=[
            pl.BlockSpec((_PB, 128), lambda i: (i, 0)),
            pl.BlockSpec((_PB, 128), lambda i: (i, 0)),
        ],
        out_shape=[
            jax.ShapeDtypeStruct((PK, 128), jnp.float32),
            jax.ShapeDtypeStruct((PK, 128), jnp.float32),
        ],
    )(y0p, agg1p, b1p, bd20, bd21)

    agg2 = _sc_agg(rowp, colp, wn, z1p.reshape(NP, HID))
    agg2p = agg2.reshape(NCORES, PK, 128)

    out = pl.pallas_call(
        _fin_body,
        grid=(grid,),
        in_specs=[
            pl.BlockSpec((_PB, 128), lambda i: (i, 0)),
            pl.BlockSpec((NCORES, _PB, 128), lambda i: (0, i, 0)),
            pl.BlockSpec((8, 128), lambda i: (0, 0)),
            pl.BlockSpec((128, 128), lambda i: (0, 0)),
        ],
        out_specs=pl.BlockSpec((_PB, 128), lambda i: (i, 0)),
        out_shape=jax.ShapeDtypeStruct((PK, 128), jnp.float32),
    )(z0p, agg2p, b2p, sg)
    return out.reshape(NP, C_OUT)[:N]


# speedup vs baseline: 1.5425x; 1.2262x over previous
import functools

import jax
import jax.numpy as jnp
from jax import lax
from jax.experimental import pallas as pl
from jax.experimental.pallas import tpu as pltpu
from jax.experimental.pallas import tpu_sc as plsc

N = 10000
NP = 10240
E = 320000
NCORES = 2
NSUB = 16
NTILES = NCORES * NSUB
CHUNKS = 80
CW = 128
EP = NTILES * CHUNKS * CW
F_IN = 128
HID = 16
C_OUT = 16
SLICE_PER_SUB = NP // NSUB
NBUF = 8

_sc_mesh = plsc.VectorSubcoreMesh(core_axis_name="c", subcore_axis_name="s")
_sc_params = pltpu.CompilerParams(
    needs_layout_passes=False, use_tc_tiling_on_sc=False)


def _rsqrt16(d):
    i = jnp.int32(0x5F3759DF) - (plsc.bitcast(d, jnp.int32) >> 1)
    y = plsc.bitcast(i, jnp.float32)
    for _ in range(3):
        y = y * (1.5 - 0.5 * d * y * y)
    return y


def _zero_rows(ref, n):
    z16 = jnp.zeros((16,), jnp.float32)

    def body(i, _):
        ref[i] = z16
        return 0

    lax.fori_loop(0, n, body, 0)


def _zero_flat(ref, n16):
    z16 = jnp.zeros((16,), jnp.float32)

    def body(i, _):
        ref[pl.ds(i * 16, 16)] = z16
        return 0

    lax.fori_loop(0, n16, body, 0)


def _agg_ring(colb, wb, row_of, nch, v_hbm, sh_agg, rows_v, sems):
    for b in range(NBUF):
        pltpu.async_copy(v_hbm.at[colb.at[b]], rows_v.at[b], sems.at[b])

    def body(i, _):
        for b in range(NBUF):
            jj = i * NBUF + b
            pltpu.make_async_copy(
                v_hbm.at[colb.at[jj]], rows_v.at[b], sems.at[b]).wait()
            for k in range(CW // 16):
                w16 = wb[jj, pl.ds(k * 16, 16)]
                base = k * 16
                for g in range(16):
                    wg = jnp.broadcast_to(w16[g], (16,))
                    rows_v[b, base + g] = rows_v[b, base + g] * wg
            pltpu.sync_copy(rows_v.at[b], sh_agg.at[row_of(jj)], add=True)

            @pl.when(jj + NBUF < nch)
            def _():
                pltpu.async_copy(
                    v_hbm.at[colb.at[jj + NBUF]], rows_v.at[b], sems.at[b])
        return 0

    lax.fori_loop(0, nch // NBUF, body, 0)


def _agg_writeout(c, s, sh_agg, agg_hbm):
    def body(j, _):
        r0 = s * SLICE_PER_SUB + j * CW
        pltpu.sync_copy(sh_agg.at[pl.ds(r0, CW)], agg_hbm.at[c, pl.ds(r0, CW)])
        return 0

    lax.fori_loop(0, SLICE_PER_SUB // CW, body, 0)


@functools.partial(
    pl.kernel,
    out_type=jax.ShapeDtypeStruct((NTILES, CHUNKS, CW), jnp.float32),
    mesh=_sc_mesh,
    compiler_params=_sc_params,
    scratch_types=(
        pltpu.VMEM((2, CHUNKS, CW), jnp.int32),
        pltpu.VMEM((2, CHUNKS, CW), jnp.float32),
        pltpu.VMEM((CHUNKS, CW), jnp.int32),
        pltpu.VMEM((CHUNKS, CW), jnp.float32),
        pltpu.VMEM((NP,), jnp.float32),
        pltpu.VMEM((NP,), jnp.float32),
        pltpu.VMEM((SLICE_PER_SUB,), jnp.float32),
        pltpu.VMEM((SLICE_PER_SUB,), jnp.float32),
        pltpu.VMEM((SLICE_PER_SUB,), jnp.float32),
        pltpu.VMEM_SHARED((NSUB, NP), jnp.float32),
        pltpu.VMEM_SHARED((NP,), jnp.float32),
    ),
)
def _sc_prep(row_hbm, col_hbm, ew_hbm, wn_hbm,
             row2, ew2, colb, wb, degl, disb, acc, tmpd, disc,
             sh_slots, sh_dis):
    c = lax.axis_index("c")
    s = lax.axis_index("s")
    wid = c * NSUB + s

    pltpu.sync_copy(row_hbm.at[s], row2.at[0])
    pltpu.sync_copy(row_hbm.at[s + NSUB], row2.at[1])
    pltpu.sync_copy(ew_hbm.at[s], ew2.at[0])
    pltpu.sync_copy(ew_hbm.at[s + NSUB], ew2.at[1])
    pltpu.sync_copy(col_hbm.at[wid], colb)

    _zero_flat(degl, NP // 16)

    def deg_body(j, _):
        for t in range(2):
            for k in range(8):
                sl = pl.ds(k * 16, 16)
                plsc.addupdate_scatter(degl, [row2[t, j, sl]], ew2[t, j, sl])
        return 0

    lax.fori_loop(0, CHUNKS, deg_body, 0)

    pltpu.sync_copy(degl, sh_slots.at[s])
    plsc.subcore_barrier()

    base = s * SLICE_PER_SUB
    _zero_flat(acc, SLICE_PER_SUB // 16)

    def red_body(t, _):
        pltpu.sync_copy(sh_slots.at[t, pl.ds(base, SLICE_PER_SUB)], tmpd)

        def add_body(r, _):
            sl = pl.ds(r * 16, 16)
            acc[sl] = acc[sl] + tmpd[sl]
            return 0

        lax.fori_loop(0, SLICE_PER_SUB // 16, add_body, 0)
        return 0

    lax.fori_loop(0, NSUB, red_body, 0)

    def dis_body(r, _):
        sl = pl.ds(r * 16, 16)
        d = acc[sl]
        y = _rsqrt16(jnp.maximum(d, 1e-30))
        disc[sl] = jnp.where(d > 0, y, 0.0)
        return 0

    lax.fori_loop(0, SLICE_PER_SUB // 16, dis_body, 0)
    pltpu.sync_copy(disc, sh_dis.at[pl.ds(base, SLICE_PER_SUB)])
    plsc.subcore_barrier()
    pltpu.sync_copy(sh_dis, disb)

    def wn_body(j, _):
        for k in range(8):
            sl = pl.ds(k * 16, 16)
            dr = plsc.load_gather(disb, [row2[c, j, sl]])
            dc = plsc.load_gather(disb, [colb[j, sl]])
            wb[j, sl] = dr * ew2[c, j, sl] * dc
        return 0

    lax.fori_loop(0, CHUNKS, wn_body, 0)
    pltpu.sync_copy(wb, wn_hbm.at[wid])


@functools.partial(
    pl.kernel,
    out_type=jax.ShapeDtypeStruct((NCORES, NP, HID), jnp.float32),
    mesh=_sc_mesh,
    compiler_params=_sc_params,
    scratch_types=(
        pltpu.VMEM((CHUNKS, CW), jnp.int32),
        pltpu.VMEM((CHUNKS, CW), jnp.int32),
        pltpu.VMEM((CHUNKS, CW), jnp.float32),
        pltpu.VMEM((CW, 16), jnp.float32),
        pltpu.VMEM((NBUF, CW, HID), jnp.float32),
        pltpu.VMEM_SHARED((NP, HID), jnp.float32),
        pltpu.VMEM_SHARED((NP, HID), jnp.float32),
        pltpu.SemaphoreType.DMA((NBUF,)),
    ),
)
def _sc_agg(row_hbm, col_hbm, wn_hbm, v_hbm, agg_hbm,
            rowb, colb, wb, zb, rows_v, sh_agg, sh_v, sems):
    c = lax.axis_index("c")
    s = lax.axis_index("s")
    wid = c * NSUB + s

    pltpu.sync_copy(row_hbm.at[wid], rowb)
    pltpu.sync_copy(col_hbm.at[wid], colb)
    pltpu.sync_copy(wn_hbm.at[wid], wb)
    sl_v = pl.ds(s * SLICE_PER_SUB, SLICE_PER_SUB)
    pltpu.sync_copy(v_hbm.at[sl_v], sh_v.at[sl_v])
    _zero_rows(zb, CW)

    def za_body(j, _):
        pltpu.sync_copy(zb, sh_agg.at[pl.ds(s * SLICE_PER_SUB + j * CW, CW)])
        return 0

    lax.fori_loop(0, SLICE_PER_SUB // CW, za_body, 0)
    plsc.subcore_barrier()

    _agg_ring(colb, wb, lambda jj: rowb.at[jj], CHUNKS,
              sh_v, sh_agg, rows_v, sems)
    plsc.subcore_barrier()
    _agg_writeout(c, s, sh_agg, agg_hbm)


def _mm1_body(x_ref, w0_ref, w1_ref, y0_ref, y1_ref):
    x = x_ref[...]
    y0_ref[...] = jnp.dot(x, w0_ref[...], preferred_element_type=jnp.float32)
    y1_ref[...] = jnp.dot(x, w1_ref[...], preferred_element_type=jnp.float32)


def _mid_body(y0_ref, agg_ref, b_ref, w0_ref, w1_ref, z0_ref, z1_ref):
    h = jnp.maximum(y0_ref[...] - (agg_ref[0] + agg_ref[1]) + b_ref[0:1, :],
                    0.0)
    z0_ref[...] = jnp.dot(h, w0_ref[...], preferred_element_type=jnp.float32)
    z1_ref[...] = jnp.dot(h, w1_ref[...], preferred_element_type=jnp.float32)


def _fin_body(z0_ref, agg_ref, b_ref, sg_ref, out_ref):
    o = z0_ref[...] - (agg_ref[0] + agg_ref[1]) + b_ref[0:1, :]
    m = jnp.max(o, axis=1, keepdims=True)
    ex = jnp.exp(o - m)
    gs = jnp.dot(ex, sg_ref[...], preferred_element_type=jnp.float32)
    out_ref[...] = o - m - jnp.log(gs)


_RB = 1024


def kernel(x, edge_index, edge_weight, W1, b1, W2, b2):
    row = edge_index[0]
    col = edge_index[1]
    pad = EP - E
    zpad_i = jnp.zeros((pad,), row.dtype)
    rowp = jnp.concatenate([row, zpad_i]).reshape(NTILES, CHUNKS, CW)
    colp = jnp.concatenate([col, zpad_i]).reshape(NTILES, CHUNKS, CW)
    ewp = jnp.concatenate([edge_weight, jnp.zeros((pad,), edge_weight.dtype)])
    ewp = ewp.reshape(NTILES, CHUNKS, CW)
    b1p = jnp.broadcast_to(jnp.tile(b1, 8).reshape(1, 128), (8, 128))
    b2p = jnp.broadcast_to(jnp.tile(b2, 8).reshape(1, 128), (8, 128))
    eye8 = jnp.eye(8, dtype=jnp.float32)
    bd20 = jnp.kron(eye8, W2[0])
    bd21 = jnp.kron(eye8, W2[1])
    sg = jnp.kron(eye8, jnp.ones((C_OUT, C_OUT), jnp.float32))
    xp = jnp.pad(x, ((0, NP - N), (0, 0)))

    grid = NP // _RB
    y0, y1 = pl.pallas_call(
        _mm1_body,
        grid=(grid,),
        in_specs=[
            pl.BlockSpec((_RB, F_IN), lambda i: (i, 0)),
            pl.BlockSpec((F_IN, HID), lambda i: (0, 0)),
            pl.BlockSpec((F_IN, HID), lambda i: (0, 0)),
        ],
        out_specs=[
            pl.BlockSpec((_RB, HID), lambda i: (i, 0)),
            pl.BlockSpec((_RB, HID), lambda i: (i, 0)),
        ],
        out_shape=[
            jax.ShapeDtypeStruct((NP, HID), jnp.float32),
            jax.ShapeDtypeStruct((NP, HID), jnp.float32),
        ],
    )(xp, W1[0], W1[1])

    wn = _sc_prep(rowp, colp, ewp)
    agg1 = _sc_agg(rowp, colp, wn, y1)

    PK = NP * HID // 128
    _PB = _RB * HID // 128
    agg1p = agg1.reshape(NCORES, PK, 128)
    y0p = y0.reshape(PK, 128)

    z0p, z1p = pl.pallas_call(
        _mid_body,
        grid=(grid,),
        in_specs=[
            pl.BlockSpec((_PB, 128), lambda i: (i, 0)),
            pl.BlockSpec((NCORES, _PB, 128), lambda i: (0, i, 0)),
            pl.BlockSpec((8, 128), lambda i: (0, 0)),
            pl.BlockSpec((128, 128), lambda i: (0, 0)),
            pl.BlockSpec((128, 128), lambda i: (0, 0)),
        ],
        out_specs=[
            pl.BlockSpec((_PB, 128), lambda i: (i, 0)),
            pl.BlockSpec((_PB, 128), lambda i: (i, 0)),
        ],
        out_shape=[
            jax.ShapeDtypeStruct((PK, 128), jnp.float32),
            jax.ShapeDtypeStruct((PK, 128), jnp.float32),
        ],
    )(y0p, agg1p, b1p, bd20, bd21)

    agg2 = _sc_agg(rowp, colp, wn, z1p.reshape(NP, HID))
    agg2p = agg2.reshape(NCORES, PK, 128)

    out = pl.pallas_call(
        _fin_body,
        grid=(grid,),
        in_specs=[
            pl.BlockSpec((_PB, 128), lambda i: (i, 0)),
            pl.BlockSpec((NCORES, _PB, 128), lambda i: (0, i, 0)),
            pl.BlockSpec((8, 128), lambda i: (0, 0)),
            pl.BlockSpec((128, 128), lambda i: (0, 0)),
        ],
        out_specs=pl.BlockSpec((_PB, 128), lambda i: (i, 0)),
        out_shape=jax.ShapeDtypeStruct((PK, 128), jnp.float32),
    )(z0p, agg2p, b2p, sg)
    return out.reshape(NP, C_OUT)[:N]


# speedup vs baseline: 1.6074x; 1.0421x over previous
import functools

import jax
import jax.numpy as jnp
from jax import lax
from jax.experimental import pallas as pl
from jax.experimental.pallas import tpu as pltpu
from jax.experimental.pallas import tpu_sc as plsc

N = 10000
NP = 10240
E = 320000
NCORES = 2
NSUB = 16
NTILES = NCORES * NSUB
CHUNKS = 80
CW = 128
EP = NTILES * CHUNKS * CW
F_IN = 128
HID = 16
C_OUT = 16
SLICE_PER_SUB = NP // NSUB
NBUF = 8

_sc_mesh = plsc.VectorSubcoreMesh(core_axis_name="c", subcore_axis_name="s")
_sc_params = pltpu.CompilerParams(
    needs_layout_passes=False, use_tc_tiling_on_sc=False)


def _rsqrt16(d):
    i = jnp.int32(0x5F3759DF) - (plsc.bitcast(d, jnp.int32) >> 1)
    y = plsc.bitcast(i, jnp.float32)
    for _ in range(3):
        y = y * (1.5 - 0.5 * d * y * y)
    return y


def _zero_rows(ref, n):
    z16 = jnp.zeros((16,), jnp.float32)

    def body(i, _):
        ref[i] = z16
        return 0

    lax.fori_loop(0, n, body, 0)


def _zero_flat(ref, n16):
    z16 = jnp.zeros((16,), jnp.float32)

    def body(i, _):
        ref[pl.ds(i * 16, 16)] = z16
        return 0

    lax.fori_loop(0, n16, body, 0)


def _agg_ring(colb, wb, row_of, nch, v_hbm, sh_agg, rows_v, sems):
    for b in range(NBUF):
        pltpu.async_copy(v_hbm.at[colb.at[b]], rows_v.at[b], sems.at[b])

    def body(i, _):
        for b in range(NBUF):
            jj = i * NBUF + b
            pltpu.make_async_copy(
                v_hbm.at[colb.at[jj]], rows_v.at[b], sems.at[b]).wait()
            for k in range(CW // 16):
                w16 = wb[jj, pl.ds(k * 16, 16)]
                base = k * 16
                for g in range(16):
                    wg = jnp.broadcast_to(w16[g], (16,))
                    rows_v[b, base + g] = rows_v[b, base + g] * wg
            pltpu.sync_copy(rows_v.at[b], sh_agg.at[row_of(jj)], add=True)

            @pl.when(jj + NBUF < nch)
            def _():
                pltpu.async_copy(
                    v_hbm.at[colb.at[jj + NBUF]], rows_v.at[b], sems.at[b])
        return 0

    lax.fori_loop(0, nch // NBUF, body, 0)


def _agg_writeout(c, s, sh_agg, agg_hbm):
    def body(j, _):
        r0 = s * SLICE_PER_SUB + j * CW
        pltpu.sync_copy(sh_agg.at[pl.ds(r0, CW)], agg_hbm.at[c, pl.ds(r0, CW)])
        return 0

    lax.fori_loop(0, SLICE_PER_SUB // CW, body, 0)


@functools.partial(
    pl.kernel,
    out_type=jax.ShapeDtypeStruct((NTILES, CHUNKS, CW), jnp.float32),
    mesh=_sc_mesh,
    compiler_params=_sc_params,
    scratch_types=(
        pltpu.VMEM((2, CHUNKS, CW), jnp.int32),
        pltpu.VMEM((2, CHUNKS, CW), jnp.float32),
        pltpu.VMEM((CHUNKS, CW), jnp.int32),
        pltpu.VMEM((CHUNKS, CW), jnp.float32),
        pltpu.VMEM((NP,), jnp.float32),
        pltpu.VMEM((NP,), jnp.float32),
        pltpu.VMEM((NSUB, SLICE_PER_SUB), jnp.float32),
        pltpu.VMEM((SLICE_PER_SUB,), jnp.float32),
        pltpu.VMEM_SHARED((NSUB, NP), jnp.float32),
        pltpu.VMEM_SHARED((NP,), jnp.float32),
        pltpu.SemaphoreType.DMA,
    ),
)
def _sc_prep(row_hbm, col_hbm, ew_hbm, wn_hbm,
             row2, ew2, colb, wb, degl, disb, tmps, disc,
             sh_slots, sh_dis, sem):
    c = lax.axis_index("c")
    s = lax.axis_index("s")
    wid = c * NSUB + s

    d1 = pltpu.async_copy(row_hbm.at[s], row2.at[0], sem)
    d2 = pltpu.async_copy(row_hbm.at[s + NSUB], row2.at[1], sem)
    d3 = pltpu.async_copy(ew_hbm.at[s], ew2.at[0], sem)
    d4 = pltpu.async_copy(ew_hbm.at[s + NSUB], ew2.at[1], sem)
    d5 = pltpu.async_copy(col_hbm.at[wid], colb, sem)

    _zero_flat(degl, NP // 16)
    d1.wait()
    d2.wait()
    d3.wait()
    d4.wait()
    d5.wait()

    def deg_body(j, _):
        for t in range(2):
            for k in range(8):
                sl = pl.ds(k * 16, 16)
                plsc.addupdate_scatter(degl, [row2[t, j, sl]], ew2[t, j, sl])
        return 0

    lax.fori_loop(0, CHUNKS, deg_body, 0)

    pltpu.sync_copy(degl, sh_slots.at[s])
    plsc.subcore_barrier()

    base = s * SLICE_PER_SUB
    pltpu.sync_copy(
        sh_slots.at[pl.ds(0, NSUB), pl.ds(base, SLICE_PER_SUB)], tmps)

    def dis_body(r, _):
        sl = pl.ds(r * 16, 16)
        vals = [tmps[t, sl] for t in range(NSUB)]
        while len(vals) > 1:
            vals = [vals[i] + vals[i + 1] for i in range(0, len(vals), 2)]
        d = vals[0]
        y = _rsqrt16(jnp.maximum(d, 1e-30))
        disc[sl] = jnp.where(d > 0, y, 0.0)
        return 0

    lax.fori_loop(0, SLICE_PER_SUB // 16, dis_body, 0)
    pltpu.sync_copy(disc, sh_dis.at[pl.ds(base, SLICE_PER_SUB)])
    plsc.subcore_barrier()
    pltpu.sync_copy(sh_dis, disb)

    def wn_body(j, _):
        for k in range(8):
            sl = pl.ds(k * 16, 16)
            dr = plsc.load_gather(disb, [row2[c, j, sl]])
            dc = plsc.load_gather(disb, [colb[j, sl]])
            wb[j, sl] = dr * ew2[c, j, sl] * dc
        return 0

    lax.fori_loop(0, CHUNKS, wn_body, 0)
    pltpu.sync_copy(wb, wn_hbm.at[wid])


@functools.partial(
    pl.kernel,
    out_type=jax.ShapeDtypeStruct((NCORES, NP, HID), jnp.float32),
    mesh=_sc_mesh,
    compiler_params=_sc_params,
    scratch_types=(
        pltpu.VMEM((CHUNKS, CW), jnp.int32),
        pltpu.VMEM((CHUNKS, CW), jnp.int32),
        pltpu.VMEM((CHUNKS, CW), jnp.float32),
        pltpu.VMEM((CW, 16), jnp.float32),
        pltpu.VMEM((NBUF, CW, HID), jnp.float32),
        pltpu.VMEM_SHARED((NP, HID), jnp.float32),
        pltpu.VMEM_SHARED((NP, HID), jnp.float32),
        pltpu.SemaphoreType.DMA((NBUF,)),
    ),
)
def _sc_agg(row_hbm, col_hbm, wn_hbm, v_hbm, agg_hbm,
            rowb, colb, wb, zb, rows_v, sh_agg, sh_v, sems):
    c = lax.axis_index("c")
    s = lax.axis_index("s")
    wid = c * NSUB + s

    pltpu.sync_copy(row_hbm.at[wid], rowb)
    pltpu.sync_copy(col_hbm.at[wid], colb)
    pltpu.sync_copy(wn_hbm.at[wid], wb)
    sl_v = pl.ds(s * SLICE_PER_SUB, SLICE_PER_SUB)
    pltpu.sync_copy(v_hbm.at[sl_v], sh_v.at[sl_v])
    _zero_rows(zb, CW)

    def za_body(j, _):
        pltpu.sync_copy(zb, sh_agg.at[pl.ds(s * SLICE_PER_SUB + j * CW, CW)])
        return 0

    lax.fori_loop(0, SLICE_PER_SUB // CW, za_body, 0)
    plsc.subcore_barrier()

    _agg_ring(colb, wb, lambda jj: rowb.at[jj], CHUNKS,
              sh_v, sh_agg, rows_v, sems)
    plsc.subcore_barrier()
    _agg_writeout(c, s, sh_agg, agg_hbm)


def _mm1_body(x_ref, w0_ref, w1_ref, y0_ref, y1_ref):
    x = x_ref[...]
    y0_ref[...] = jnp.dot(x, w0_ref[...], preferred_element_type=jnp.float32)
    y1_ref[...] = jnp.dot(x, w1_ref[...], preferred_element_type=jnp.float32)


def _mid_body(y0_ref, agg_ref, b_ref, w0_ref, w1_ref, z0_ref, z1_ref):
    h = jnp.maximum(y0_ref[...] - (agg_ref[0] + agg_ref[1]) + b_ref[0:1, :],
                    0.0)
    z0_ref[...] = jnp.dot(h, w0_ref[...], preferred_element_type=jnp.float32)
    z1_ref[...] = jnp.dot(h, w1_ref[...], preferred_element_type=jnp.float32)


def _fin_body(z0_ref, agg_ref, b_ref, sg_ref, out_ref):
    o = z0_ref[...] - (agg_ref[0] + agg_ref[1]) + b_ref[0:1, :]
    m = jnp.max(o, axis=1, keepdims=True)
    ex = jnp.exp(o - m)
    gs = jnp.dot(ex, sg_ref[...], preferred_element_type=jnp.float32)
    out_ref[...] = o - m - jnp.log(gs)


_RB = 1024


def kernel(x, edge_index, edge_weight, W1, b1, W2, b2):
    row = edge_index[0]
    col = edge_index[1]
    pad = EP - E
    zpad_i = jnp.zeros((pad,), row.dtype)
    rowp = jnp.concatenate([row, zpad_i]).reshape(NTILES, CHUNKS, CW)
    colp = jnp.concatenate([col, zpad_i]).reshape(NTILES, CHUNKS, CW)
    ewp = jnp.concatenate([edge_weight, jnp.zeros((pad,), edge_weight.dtype)])
    ewp = ewp.reshape(NTILES, CHUNKS, CW)
    b1p = jnp.broadcast_to(jnp.tile(b1, 8).reshape(1, 128), (8, 128))
    b2p = jnp.broadcast_to(jnp.tile(b2, 8).reshape(1, 128), (8, 128))
    eye8 = jnp.eye(8, dtype=jnp.float32)
    bd20 = jnp.kron(eye8, W2[0])
    bd21 = jnp.kron(eye8, W2[1])
    sg = jnp.kron(eye8, jnp.ones((C_OUT, C_OUT), jnp.float32))
    xp = jnp.pad(x, ((0, NP - N), (0, 0)))

    grid = NP // _RB
    y0, y1 = pl.pallas_call(
        _mm1_body,
        grid=(grid,),
        in_specs=[
            pl.BlockSpec((_RB, F_IN), lambda i: (i, 0)),
            pl.BlockSpec((F_IN, HID), lambda i: (0, 0)),
            pl.BlockSpec((F_IN, HID), lambda i: (0, 0)),
        ],
        out_specs=[
            pl.BlockSpec((_RB, HID), lambda i: (i, 0)),
            pl.BlockSpec((_RB, HID), lambda i: (i, 0)),
        ],
        out_shape=[
            jax.ShapeDtypeStruct((NP, HID), jnp.float32),
            jax.ShapeDtypeStruct((NP, HID), jnp.float32),
        ],
    )(xp, W1[0], W1[1])

    wn = _sc_prep(rowp, colp, ewp)
    agg1 = _sc_agg(rowp, colp, wn, y1)

    PK = NP * HID // 128
    _PB = _RB * HID // 128
    agg1p = agg1.reshape(NCORES, PK, 128)
    y0p = y0.reshape(PK, 128)

    z0p, z1p = pl.pallas_call(
        _mid_body,
        grid=(grid,),
        in_specs=[
            pl.BlockSpec((_PB, 128), lambda i: (i, 0)),
            pl.BlockSpec((NCORES, _PB, 128), lambda i: (0, i, 0)),
            pl.BlockSpec((8, 128), lambda i: (0, 0)),
            pl.BlockSpec((128, 128), lambda i: (0, 0)),
            pl.BlockSpec((128, 128), lambda i: (0, 0)),
        ],
        out_specs=[
            pl.BlockSpec((_PB, 128), lambda i: (i, 0)),
            pl.BlockSpec((_PB, 128), lambda i: (i, 0)),
        ],
        out_shape=[
            jax.ShapeDtypeStruct((PK, 128), jnp.float32),
            jax.ShapeDtypeStruct((PK, 128), jnp.float32),
        ],
    )(y0p, agg1p, b1p, bd20, bd21)

    agg2 = _sc_agg(rowp, colp, wn, z1p.reshape(NP, HID))
    agg2p = agg2.reshape(NCORES, PK, 128)

    out = pl.pallas_call(
        _fin_body,
        grid=(grid,),
        in_specs=[
            pl.BlockSpec((_PB, 128), lambda i: (i, 0)),
            pl.BlockSpec((NCORES, _PB, 128), lambda i: (0, i, 0)),
            pl.BlockSpec((8, 128), lambda i: (0, 0)),
            pl.BlockSpec((128, 128), lambda i: (0, 0)),
        ],
        out_specs=pl.BlockSpec((_PB, 128), lambda i: (i, 0)),
        out_shape=jax.ShapeDtypeStruct((PK, 128), jnp.float32),
    )(z0p, agg2p, b2p, sg)
    return out.reshape(NP, C_OUT)[:N]


# speedup vs baseline: 2.0712x; 1.2885x over previous
import functools

import jax
import jax.numpy as jnp
from jax import lax
from jax.experimental import pallas as pl
from jax.experimental.pallas import tpu as pltpu
from jax.experimental.pallas import tpu_sc as plsc

N = 10000
NP = 10240
E = 320000
NCORES = 2
NSUB = 16
NTILES = NCORES * NSUB
CW = 128
TCH = E // CW
BCH = TCH // NTILES
XN = TCH - NTILES * BCH
DCH = TCH // NSUB
DXN = TCH - NSUB * DCH
F_IN = 128
HID = 16
C_OUT = 16
SLICE_PER_SUB = NP // NSUB
NBUF = 6

_sc_mesh = plsc.VectorSubcoreMesh(core_axis_name="c", subcore_axis_name="s")
_sc_params = pltpu.CompilerParams(
    needs_layout_passes=False, use_tc_tiling_on_sc=False)


def _rsqrt16(d):
    i = jnp.int32(0x5F3759DF) - (plsc.bitcast(d, jnp.int32) >> 1)
    y = plsc.bitcast(i, jnp.float32)
    for _ in range(3):
        y = y * (1.5 - 0.5 * d * y * y)
    return y


def _zero_rows(ref, n):
    z16 = jnp.zeros((16,), jnp.float32)

    def body(i, _):
        ref[i] = z16
        return 0

    lax.fori_loop(0, n, body, 0)


def _zero_flat(ref, n16):
    z16 = jnp.zeros((16,), jnp.float32)

    def body(i, _):
        ref[pl.ds(i * 16, 16)] = z16
        return 0

    lax.fori_loop(0, n16, body, 0)


def _agg_ring(colb, wb, row_of, nch, v_hbm, sh_agg, rows_v, sems):
    for b in range(NBUF):
        pltpu.async_copy(v_hbm.at[colb.at[b]], rows_v.at[b], sems.at[b])

    def body(i, _):
        for b in range(NBUF):
            jj = i * NBUF + b
            pltpu.make_async_copy(
                v_hbm.at[colb.at[jj]], rows_v.at[b], sems.at[b]).wait()
            for k in range(CW // 16):
                w16 = wb[jj, pl.ds(k * 16, 16)]
                base = k * 16
                for g in range(16):
                    wg = jnp.broadcast_to(w16[g], (16,))
                    rows_v[b, base + g] = rows_v[b, base + g] * wg
            pltpu.sync_copy(rows_v.at[b], sh_agg.at[row_of(jj)], add=True)

            @pl.when(jj + NBUF < nch)
            def _():
                pltpu.async_copy(
                    v_hbm.at[colb.at[jj + NBUF]], rows_v.at[b], sems.at[b])
        return 0

    lax.fori_loop(0, nch // NBUF, body, 0)


def _agg_writeout(c, s, sh_agg, agg_hbm):
    def body(j, _):
        r0 = s * SLICE_PER_SUB + j * CW
        pltpu.sync_copy(sh_agg.at[pl.ds(r0, CW)], agg_hbm.at[c, pl.ds(r0, CW)])
        return 0

    lax.fori_loop(0, SLICE_PER_SUB // CW, body, 0)


@functools.partial(
    pl.kernel,
    out_type=jax.ShapeDtypeStruct((TCH, CW), jnp.float32),
    mesh=_sc_mesh,
    compiler_params=_sc_params,
    scratch_types=(
        pltpu.VMEM((DCH + 1, CW), jnp.int32),
        pltpu.VMEM((DCH + 1, CW), jnp.float32),
        pltpu.VMEM((BCH + 1, CW), jnp.int32),
        pltpu.VMEM((BCH + 1, CW), jnp.int32),
        pltpu.VMEM((BCH + 1, CW), jnp.float32),
        pltpu.VMEM((BCH + 1, CW), jnp.float32),
        pltpu.VMEM((NP,), jnp.float32),
        pltpu.VMEM((NP,), jnp.float32),
        pltpu.VMEM((NSUB, SLICE_PER_SUB), jnp.float32),
        pltpu.VMEM((SLICE_PER_SUB,), jnp.float32),
        pltpu.VMEM_SHARED((NSUB, NP), jnp.float32),
        pltpu.VMEM_SHARED((NP,), jnp.float32),
        pltpu.SemaphoreType.DMA,
    ),
)
def _sc_prep(ei_hbm, ew_hbm, wn_hbm,
             rowD, ewD, rowW, colW, ewW, wb, degl, disb, tmps, disc,
             sh_slots, sh_dis, sem):
    c = lax.axis_index("c")
    s = lax.axis_index("s")
    wid = c * NSUB + s
    dbase = s * DCH
    wbase = wid * BCH

    d1 = pltpu.async_copy(ei_hbm.at[0, pl.ds(dbase, DCH)],
                          rowD.at[pl.ds(0, DCH)], sem)
    d2 = pltpu.async_copy(ew_hbm.at[pl.ds(dbase, DCH)],
                          ewD.at[pl.ds(0, DCH)], sem)
    d3 = pltpu.async_copy(ei_hbm.at[0, pl.ds(wbase, BCH)],
                          rowW.at[pl.ds(0, BCH)], sem)
    d4 = pltpu.async_copy(ei_hbm.at[1, pl.ds(wbase, BCH)],
                          colW.at[pl.ds(0, BCH)], sem)
    d5 = pltpu.async_copy(ew_hbm.at[pl.ds(wbase, BCH)],
                          ewW.at[pl.ds(0, BCH)], sem)

    @pl.when(s < DXN)
    def _():
        pltpu.sync_copy(ei_hbm.at[0, pl.ds(NSUB * DCH + s, 1)],
                        rowD.at[pl.ds(DCH, 1)])
        pltpu.sync_copy(ew_hbm.at[pl.ds(NSUB * DCH + s, 1)],
                        ewD.at[pl.ds(DCH, 1)])

    @pl.when(wid < XN)
    def _():
        xc = NTILES * BCH + wid
        pltpu.sync_copy(ei_hbm.at[0, pl.ds(xc, 1)], rowW.at[pl.ds(BCH, 1)])
        pltpu.sync_copy(ei_hbm.at[1, pl.ds(xc, 1)], colW.at[pl.ds(BCH, 1)])
        pltpu.sync_copy(ew_hbm.at[pl.ds(xc, 1)], ewW.at[pl.ds(BCH, 1)])

    _zero_flat(degl, NP // 16)
    d1.wait()
    d2.wait()
    d3.wait()
    d4.wait()
    d5.wait()

    def deg_body(j, _):
        for k in range(8):
            sl = pl.ds(k * 16, 16)
            plsc.addupdate_scatter(degl, [rowD[j, sl]], ewD[j, sl])
        return 0

    lax.fori_loop(0, DCH, deg_body, 0)

    @pl.when(s < DXN)
    def _():
        for k in range(8):
            sl = pl.ds(k * 16, 16)
            plsc.addupdate_scatter(degl, [rowD[DCH, sl]], ewD[DCH, sl])

    pltpu.sync_copy(degl, sh_slots.at[s])
    plsc.subcore_barrier()

    base = s * SLICE_PER_SUB
    pltpu.sync_copy(
        sh_slots.at[pl.ds(0, NSUB), pl.ds(base, SLICE_PER_SUB)], tmps)

    def dis_body(r, _):
        sl = pl.ds(r * 16, 16)
        vals = [tmps[t, sl] for t in range(NSUB)]
        while len(vals) > 1:
            vals = [vals[i] + vals[i + 1] for i in range(0, len(vals), 2)]
        d = vals[0]
        y = _rsqrt16(jnp.maximum(d, 1e-30))
        disc[sl] = jnp.where(d > 0, y, 0.0)
        return 0

    lax.fori_loop(0, SLICE_PER_SUB // 16, dis_body, 0)
    pltpu.sync_copy(disc, sh_dis.at[pl.ds(base, SLICE_PER_SUB)])
    plsc.subcore_barrier()
    pltpu.sync_copy(sh_dis, disb)

    def wn_body(j, _):
        for k in range(8):
            sl = pl.ds(k * 16, 16)
            dr = plsc.load_gather(disb, [rowW[j, sl]])
            dc = plsc.load_gather(disb, [colW[j, sl]])
            wb[j, sl] = dr * ewW[j, sl] * dc
        return 0

    lax.fori_loop(0, BCH, wn_body, 0)

    @pl.when(wid < XN)
    def _():
        for k in range(8):
            sl = pl.ds(k * 16, 16)
            dr = plsc.load_gather(disb, [rowW[BCH, sl]])
            dc = plsc.load_gather(disb, [colW[BCH, sl]])
            wb[BCH, sl] = dr * ewW[BCH, sl] * dc

    pltpu.sync_copy(wb.at[pl.ds(0, BCH)], wn_hbm.at[pl.ds(wbase, BCH)])

    @pl.when(wid < XN)
    def _():
        pltpu.sync_copy(wb.at[pl.ds(BCH, 1)],
                        wn_hbm.at[pl.ds(NTILES * BCH + wid, 1)])


@functools.partial(
    pl.kernel,
    out_type=jax.ShapeDtypeStruct((NCORES, NP, HID), jnp.float32),
    mesh=_sc_mesh,
    compiler_params=_sc_params,
    scratch_types=(
        pltpu.VMEM((BCH + 1, CW), jnp.int32),
        pltpu.VMEM((BCH + 1, CW), jnp.int32),
        pltpu.VMEM((BCH + 1, CW), jnp.float32),
        pltpu.VMEM((CW, 16), jnp.float32),
        pltpu.VMEM((NBUF, CW, HID), jnp.float32),
        pltpu.VMEM_SHARED((NP, HID), jnp.float32),
        pltpu.VMEM_SHARED((NP, HID), jnp.float32),
        pltpu.SemaphoreType.DMA((NBUF,)),
    ),
)
def _sc_agg(ei_hbm, wn_hbm, v_hbm, agg_hbm,
            rowb, colb, wb, zb, rows_v, sh_agg, sh_v, sems):
    c = lax.axis_index("c")
    s = lax.axis_index("s")
    wid = c * NSUB + s
    wbase = wid * BCH

    d1 = pltpu.async_copy(ei_hbm.at[0, pl.ds(wbase, BCH)],
                          rowb.at[pl.ds(0, BCH)], sems.at[0])
    d2 = pltpu.async_copy(ei_hbm.at[1, pl.ds(wbase, BCH)],
                          colb.at[pl.ds(0, BCH)], sems.at[0])
    d3 = pltpu.async_copy(wn_hbm.at[pl.ds(wbase, BCH)],
                          wb.at[pl.ds(0, BCH)], sems.at[0])
    sl_v = pl.ds(s * SLICE_PER_SUB, SLICE_PER_SUB)
    d4 = pltpu.async_copy(v_hbm.at[sl_v], sh_v.at[sl_v], sems.at[0])

    @pl.when(wid < XN)
    def _():
        xc = NTILES * BCH + wid
        pltpu.sync_copy(ei_hbm.at[0, pl.ds(xc, 1)], rowb.at[pl.ds(BCH, 1)])
        pltpu.sync_copy(ei_hbm.at[1, pl.ds(xc, 1)], colb.at[pl.ds(BCH, 1)])
        pltpu.sync_copy(wn_hbm.at[pl.ds(xc, 1)], wb.at[pl.ds(BCH, 1)])

    _zero_rows(zb, CW)
    d1.wait()
    d2.wait()
    d3.wait()
    d4.wait()

    def za_body(j, _):
        pltpu.sync_copy(zb, sh_agg.at[pl.ds(s * SLICE_PER_SUB + j * CW, CW)])
        return 0

    lax.fori_loop(0, SLICE_PER_SUB // CW, za_body, 0)
    plsc.subcore_barrier()

    _agg_ring(colb, wb, lambda jj: rowb.at[jj], BCH,
              sh_v, sh_agg, rows_v, sems)

    @pl.when(wid < XN)
    def _():
        pltpu.async_copy(
            sh_v.at[colb.at[BCH]], rows_v.at[0], sems.at[0]).wait()
        for k in range(CW // 16):
            w16 = wb[BCH, pl.ds(k * 16, 16)]
            base = k * 16
            for g in range(16):
                wg = jnp.broadcast_to(w16[g], (16,))
                rows_v[0, base + g] = rows_v[0, base + g] * wg
        pltpu.sync_copy(rows_v.at[0], sh_agg.at[rowb.at[BCH]], add=True)

    plsc.subcore_barrier()
    _agg_writeout(c, s, sh_agg, agg_hbm)


def _mm1_body(x_ref, w0_ref, w1_ref, y0_ref, y1_ref):
    x = x_ref[...]
    y0_ref[...] = jnp.dot(x, w0_ref[...], preferred_element_type=jnp.float32)
    y1_ref[...] = jnp.dot(x, w1_ref[...], preferred_element_type=jnp.float32)


def _mid_body(y0_ref, agg_ref, b_ref, w0_ref, w1_ref, z0_ref, z1_ref):
    h = jnp.maximum(y0_ref[...] - (agg_ref[0] + agg_ref[1]) + b_ref[0:1, :],
                    0.0)
    z0_ref[...] = jnp.dot(h, w0_ref[...], preferred_element_type=jnp.float32)
    z1_ref[...] = jnp.dot(h, w1_ref[...], preferred_element_type=jnp.float32)


def _fin_body(z0_ref, agg_ref, b_ref, sg_ref, out_ref):
    o = z0_ref[...] - (agg_ref[0] + agg_ref[1]) + b_ref[0:1, :]
    m = jnp.max(o, axis=1, keepdims=True)
    ex = jnp.exp(o - m)
    gs = jnp.dot(ex, sg_ref[...], preferred_element_type=jnp.float32)
    out_ref[...] = o - m - jnp.log(gs)


_RB = 1024


def kernel(x, edge_index, edge_weight, W1, b1, W2, b2):
    eiv = edge_index.reshape(2, TCH, CW)
    ewv = edge_weight.reshape(TCH, CW)
    b1p = jnp.broadcast_to(jnp.tile(b1, 8).reshape(1, 128), (8, 128))
    b2p = jnp.broadcast_to(jnp.tile(b2, 8).reshape(1, 128), (8, 128))
    eye8 = jnp.eye(8, dtype=jnp.float32)
    bd20 = jnp.kron(eye8, W2[0])
    bd21 = jnp.kron(eye8, W2[1])
    sg = jnp.kron(eye8, jnp.ones((C_OUT, C_OUT), jnp.float32))
    xp = jnp.pad(x, ((0, NP - N), (0, 0)))

    grid = NP // _RB
    y0, y1 = pl.pallas_call(
        _mm1_body,
        grid=(grid,),
        in_specs=[
            pl.BlockSpec((_RB, F_IN), lambda i: (i, 0)),
            pl.BlockSpec((F_IN, HID), lambda i: (0, 0)),
            pl.BlockSpec((F_IN, HID), lambda i: (0, 0)),
        ],
        out_specs=[
            pl.BlockSpec((_RB, HID), lambda i: (i, 0)),
            pl.BlockSpec((_RB, HID), lambda i: (i, 0)),
        ],
        out_shape=[
            jax.ShapeDtypeStruct((NP, HID), jnp.float32),
            jax.ShapeDtypeStruct((NP, HID), jnp.float32),
        ],
    )(xp, W1[0], W1[1])

    wn = _sc_prep(eiv, ewv)
    agg1 = _sc_agg(eiv, wn, y1)

    PK = NP * HID // 128
    _PB = _RB * HID // 128
    agg1p = agg1.reshape(NCORES, PK, 128)
    y0p = y0.reshape(PK, 128)

    z0p, z1p = pl.pallas_call(
        _mid_body,
        grid=(grid,),
        in_specs=[
            pl.BlockSpec((_PB, 128), lambda i: (i, 0)),
            pl.BlockSpec((NCORES, _PB, 128), lambda i: (0, i, 0)),
            pl.BlockSpec((8, 128), lambda i: (0, 0)),
            pl.BlockSpec((128, 128), lambda i: (0, 0)),
            pl.BlockSpec((128, 128), lambda i: (0, 0)),
        ],
        out_specs=[
            pl.BlockSpec((_PB, 128), lambda i: (i, 0)),
            pl.BlockSpec((_PB, 128), lambda i: (i, 0)),
        ],
        out_shape=[
            jax.ShapeDtypeStruct((PK, 128), jnp.float32),
            jax.ShapeDtypeStruct((PK, 128), jnp.float32),
        ],
    )(y0p, agg1p, b1p, bd20, bd21)

    agg2 = _sc_agg(eiv, wn, z1p.reshape(NP, HID))
    agg2p = agg2.reshape(NCORES, PK, 128)

    out = pl.pallas_call(
        _fin_body,
        grid=(grid,),
        in_specs=[
            pl.BlockSpec((_PB, 128), lambda i: (i, 0)),
            pl.BlockSpec((NCORES, _PB, 128), lambda i: (0, i, 0)),
            pl.BlockSpec((8, 128), lambda i: (0, 0)),
            pl.BlockSpec((128, 128), lambda i: (0, 0)),
        ],
        out_specs=pl.BlockSpec((_PB, 128), lambda i: (i, 0)),
        out_shape=jax.ShapeDtypeStruct((PK, 128), jnp.float32),
    )(z0p, agg2p, b2p, sg)
    return out.reshape(NP, C_OUT)[:N]
